# baseline (speedup 1.0000x reference)
import functools

import jax
import jax.numpy as jnp
from jax import lax
from jax.experimental import pallas as pl
from jax.experimental.pallas import tpu as pltpu
from jax.experimental.pallas import tpu_sc as plsc

_NC = 2
_NS = 16
_NW = _NC * _NS
_CH = 128
_SB = 1024
_BM = 256


def _round_up(a: int, m: int) -> int:
    return (a + m - 1) // m * m


@functools.lru_cache(maxsize=None)
def _build(N: int, D: int, E: int, K: int):
    P = _round_up(N, 2048)
    WPB = P // _NW
    EPAD = _round_up(E, 8) + _SB
    mesh = plsc.VectorSubcoreMesh(
        core_axis_name="c", subcore_axis_name="s",
        num_cores=_NC, num_subcores=_NS)

    @functools.partial(
        pl.kernel,
        out_type=jax.ShapeDtypeStruct((P, D), jnp.float32),
        mesh=mesh,
        scratch_types=[
            pltpu.VMEM((48,), jnp.int32),
            pltpu.VMEM((_SB,), jnp.int32),
            pltpu.VMEM((_SB,), jnp.int32),
            pltpu.VMEM((2, _CH, D), jnp.float32),
            pltpu.VMEM((WPB + 8, D), jnp.float32),
            pltpu.SemaphoreType.DMA,
            pltpu.SemaphoreType.DMA,
        ],
        compiler_params=pltpu.CompilerParams(needs_layout_passes=False),
    )
    def _segsum(starts_hbm, src_hbm, dst_hbm, hw_hbm, out_hbm,
                st_v, src_v, dst_v, stg_v, acc_v, sem0, sem1):
        c = lax.axis_index("c")
        s = lax.axis_index("s")
        w = c * _NS + s
        base = w * WPB

        zero16 = jnp.zeros((16,), jnp.float32)

        def _zb(i, carry):
            acc_v[i // (D // 16), pl.ds((i % (D // 16)) * 16, 16)] = zero16
            return carry

        lax.fori_loop(0, (WPB + 8) * (D // 16), _zb, 0)

        pltpu.sync_copy(starts_hbm, st_v)

        def _sel(idx):
            acc = jnp.zeros((), jnp.int32)
            for g in range(3):
                vec = st_v[pl.ds(g * 16, 16)]
                ids = lax.broadcasted_iota(jnp.int32, (16,), 0) + g * 16
                acc = acc + jnp.sum(jnp.where(ids == idx, vec, 0))
            return acc

        lo8 = _sel(w) & ~7
        hi8 = (_sel(w + 1) + 7) & ~7
        n_sb = lax.max((hi8 - lo8 + _SB - 1) // _SB, 0)

        def _sb_body(k, carry):
            off = pl.multiple_of(lo8 + k * _SB, 8)
            pltpu.sync_copy(src_hbm.at[pl.ds(off, _SB)], src_v)
            pltpu.sync_copy(dst_hbm.at[pl.ds(off, _SB)], dst_v)

            def _gat(cc, buf, sem):
                return pltpu.async_copy(
                    hw_hbm.at[src_v.at[pl.ds(cc * _CH, _CH)]],
                    stg_v.at[buf], sem)

            def _wait(cc, buf, sem):
                pltpu.make_async_copy(
                    hw_hbm.at[src_v.at[pl.ds(cc * _CH, _CH)]],
                    stg_v.at[buf], sem).wait()

            _gat(0, 0, sem0)
            for cc in range(_SB // _CH):
                buf = cc % 2
                sem = sem0 if buf == 0 else sem1
                if cc + 1 < _SB // _CH:
                    _gat(cc + 1, 1 - buf, sem1 if buf == 0 else sem0)
                _wait(cc, buf, sem)

                def _grp(g, carry, cc=cc, buf=buf):
                    goff = pl.multiple_of(cc * _CH + g * 16, 8)
                    tvec = dst_v[pl.ds(goff, 16)] - base
                    ok = (tvec >= 0) & (tvec < WPB)
                    tvec = jnp.where(ok, tvec, WPB)
                    rows = g * 16 + lax.broadcasted_iota(jnp.int32, (16,), 0)
                    stg2 = stg_v.at[buf]

                    def _col(cg, carry2):
                        for u in range(16):
                            cvec = jnp.full((16,), cg * 16 + u, jnp.int32)
                            vals = plsc.load_gather(stg2, [rows, cvec])
                            plsc.addupdate_scatter(acc_v, [tvec, cvec], vals)
                        return carry2

                    lax.fori_loop(0, D // 16, _col, 0)
                    return carry

                lax.fori_loop(0, _CH // 16, _grp, 0)

            return carry

        lax.fori_loop(0, n_sb, _sb_body, 0)
        pltpu.sync_copy(acc_v.at[pl.ds(0, WPB)],
                        out_hbm.at[pl.ds(base, WPB)])

    grid = (P // _BM,)
    f32 = jnp.float32

    def _row_spec():
        return pl.BlockSpec((_BM, D), lambda i: (i, 0))

    def _dis_spec():
        return pl.BlockSpec((_BM, 1), lambda i: (i, 0))

    def _full_spec(shape):
        return pl.BlockSpec(shape, lambda i: tuple(0 for _ in shape))

    def _prep_body(degp_ref, x_ref, w_ref, hw_ref, dis_ref):
        i = pl.program_id(0)
        deg = jnp.sum(degp_ref[...], axis=1) + 1.0
        dis = lax.rsqrt(jnp.maximum(deg, 1.0))[:, None]
        rows = i * _BM + lax.broadcasted_iota(jnp.int32, (_BM, 1), 0)
        dis = jnp.where(rows < N, dis, 0.0)
        dis_ref[...] = dis
        hw_ref[...] = jnp.dot(x_ref[...], w_ref[...],
                              preferred_element_type=f32) * dis

    _prep = pl.pallas_call(
        _prep_body,
        grid=grid,
        in_specs=[_row_spec(), _row_spec(), _full_spec((D, D))],
        out_specs=[_row_spec(), _dis_spec()],
        out_shape=[jax.ShapeDtypeStruct((P, D), f32),
                   jax.ShapeDtypeStruct((P, 1), f32)],
    )

    def _elu(a):
        return jnp.where(a > 0, a, jnp.exp(jnp.minimum(a, 0.0)) - 1.0)

    def _mid_first_body(p_ref, hw_ref, dis_ref, b_ref, w_ref, hwn_ref, m_ref):
        dis = dis_ref[...]
        agg = dis * (p_ref[...] + hw_ref[...]) + b_ref[...]
        h = _elu(agg)
        m_ref[...] = h
        hwn_ref[...] = jnp.dot(h, w_ref[...], preferred_element_type=f32) * dis

    _mid_first = pl.pallas_call(
        _mid_first_body,
        grid=grid,
        in_specs=[_row_spec(), _row_spec(), _dis_spec(),
                  _full_spec((1, D)), _full_spec((D, D))],
        out_specs=[_row_spec(), _row_spec()],
        out_shape=[jax.ShapeDtypeStruct((P, D), f32),
                   jax.ShapeDtypeStruct((P, D), f32)],
    )

    def _mid_body(p_ref, hw_ref, dis_ref, b_ref, w_ref, m_ref,
                  hwn_ref, mo_ref):
        dis = dis_ref[...]
        agg = dis * (p_ref[...] + hw_ref[...]) + b_ref[...]
        h = _elu(agg)
        mo_ref[...] = jnp.maximum(m_ref[...], h)
        hwn_ref[...] = jnp.dot(h, w_ref[...], preferred_element_type=f32) * dis

    _mid = pl.pallas_call(
        _mid_body,
        grid=grid,
        in_specs=[_row_spec(), _row_spec(), _dis_spec(),
                  _full_spec((1, D)), _full_spec((D, D)), _row_spec()],
        out_specs=[_row_spec(), _row_spec()],
        out_shape=[jax.ShapeDtypeStruct((P, D), f32),
                   jax.ShapeDtypeStruct((P, D), f32)],
    )

    def _fin_body(p_ref, hw_ref, dis_ref, b_ref, m_ref, out_ref):
        agg = dis_ref[...] * (p_ref[...] + hw_ref[...]) + b_ref[...]
        out_ref[...] = jnp.maximum(m_ref[...], agg)

    _fin = pl.pallas_call(
        _fin_body,
        grid=grid,
        in_specs=[_row_spec(), _row_spec(), _dis_spec(),
                  _full_spec((1, D)), _row_spec()],
        out_specs=_row_spec(),
        out_shape=jax.ShapeDtypeStruct((P, D), f32),
    )

    return P, EPAD, _segsum, _prep, _mid_first, _mid, _fin


def kernel(x, edge_index, W, b):
    N, D = x.shape
    K = W.shape[0]
    E = edge_index.shape[1]
    P, EPAD, segsum, prep, mid_first, mid, fin = _build(N, D, E, K)
    WPB = P // _NW

    x_p = jnp.pad(x, ((0, P - N), (0, 0)))
    pad_idx = jnp.full((EPAD - E,), N, jnp.int32)
    src_e = jnp.concatenate([edge_index[0], pad_idx])
    dst_e = jnp.concatenate([edge_index[1], pad_idx])
    dst_s, src_s = lax.sort((dst_e, src_e), num_keys=1)
    bounds = jnp.arange(_NW + 1, dtype=jnp.int32) * WPB
    starts = jnp.searchsorted(dst_s, bounds).astype(jnp.int32)
    starts = jnp.pad(starts, (0, 48 - _NW - 1))
    onehot = jnp.zeros((P, D), jnp.float32).at[:, 0].set(1.0)

    degp = segsum(starts, src_s, dst_s, onehot)
    hw, dis = prep(degp, x_p, W[0])
    m = None
    out = None
    for li in range(K):
        part = segsum(starts, src_s, dst_s, hw)
        bi = b[li][None]
        if li == 0:
            hw, m = mid_first(part, hw, dis, bi, W[1])
        elif li < K - 1:
            hw, m = mid(part, hw, dis, bi, W[li + 1], m)
        else:
            out = fin(part, hw, dis, bi, m)
    return out[:N]


# speedup vs baseline: 8.5004x; 8.5004x over previous
import functools

import jax
import jax.numpy as jnp
from jax import lax
from jax.experimental import pallas as pl
from jax.experimental.pallas import tpu as pltpu
from jax.experimental.pallas import tpu_sc as plsc

_NC = 2
_NS = 16
_NW = _NC * _NS
_CH = 128
_SB = 1024
_BM = 256
_GP = 128
_NOF = 80


def _round_up(a: int, m: int) -> int:
    return (a + m - 1) // m * m


@functools.lru_cache(maxsize=None)
def _build(N: int, D: int, E: int, K: int):
    P = _round_up(N, 2048)
    HP = P // 2
    WPT = HP // _NS
    APT = (HP + _GP) // _NS
    E8 = _round_up(E, 8)
    EPAD = E8 + 2 * _SB
    mesh = plsc.VectorSubcoreMesh(
        core_axis_name="c", subcore_axis_name="s",
        num_cores=_NC, num_subcores=_NS)

    @functools.partial(
        pl.kernel,
        out_type=jax.ShapeDtypeStruct((_NC, P, D), jnp.float32),
        mesh=mesh,
        scratch_types=[
            pltpu.VMEM((_NOF,), jnp.int32),
            pltpu.VMEM((_SB,), jnp.int32),
            pltpu.VMEM((_SB,), jnp.int32),
            pltpu.VMEM((_SB // _CH, _CH), jnp.int32),
            pltpu.VMEM((2, _CH, D), jnp.float32),
            pltpu.VMEM((_CH, D), jnp.float32),
            pltpu.VMEM_SHARED((HP + _GP, D), jnp.float32),
            pltpu.SemaphoreType.DMA,
            pltpu.SemaphoreType.DMA,
        ],
        compiler_params=pltpu.CompilerParams(needs_layout_passes=False),
    )
    def _segsum(ofs_hbm, src_hbm, dst_hbm, hw_hbm, out_hbm,
                of_v, src_v, dst_v, loc_v, stg_v, z_v, acc_sh, sem0, sem1):
        c = lax.axis_index("c")
        s = lax.axis_index("s")
        w = c * _NS + s

        zero16 = jnp.zeros((16,), jnp.float32)

        def _zb(i, carry):
            z_v[i // (D // 16), pl.ds((i % (D // 16)) * 16, 16)] = zero16
            return carry

        lax.fori_loop(0, _CH * (D // 16), _zb, 0)

        pltpu.sync_copy(ofs_hbm, of_v)

        def _sel(idx):
            acc = jnp.zeros((), jnp.int32)
            for g in range(_NOF // 16):
                vec = of_v[pl.ds(g * 16, 16)]
                ids = lax.broadcasted_iota(jnp.int32, (16,), 0) + g * 16
                acc = acc + jnp.sum(jnp.where(ids == idx, vec, 0))
            return acc

        for ph in range(2):
            base = ph * HP
            o1 = _sel(ph * 33 + w)
            o2 = _sel(ph * 33 + w + 1)
            n_sb = lax.max((o2 - o1 + _SB - 1) // _SB, 0)

            row0 = s * APT
            left = APT
            while left > 0:
                n = min(left, _CH)
                pltpu.sync_copy(z_v.at[pl.ds(0, n)],
                                acc_sh.at[pl.ds(row0 + (APT - left), n)])
                left -= n
            plsc.subcore_barrier()

            def _sb_body(k, carry):
                off = pl.multiple_of(o1 + k * _SB, 8)
                pltpu.sync_copy(src_hbm.at[pl.ds(off, _SB)], src_v)
                pltpu.sync_copy(dst_hbm.at[pl.ds(off, _SB)], dst_v)

                def _loc(g, carry2):
                    goff = pl.multiple_of(g * 16, 8)
                    t = dst_v[pl.ds(goff, 16)] - base
                    gpos = off + goff + lax.broadcasted_iota(
                        jnp.int32, (16,), 0)
                    ok = (t >= 0) & (t < HP) & (gpos < o2)
                    t = jnp.where(ok, t, HP + (t & (_GP - 1)))
                    loc_v[g // (_CH // 16),
                          pl.ds(pl.multiple_of((g % (_CH // 16)) * 16, 8),
                                16)] = t
                    return carry2

                lax.fori_loop(0, _SB // 16, _loc, 0)

                def _gat(cc, buf, sem):
                    return pltpu.async_copy(
                        hw_hbm.at[src_v.at[pl.ds(cc * _CH, _CH)]],
                        stg_v.at[buf], sem)

                def _wait(cc, buf, sem):
                    pltpu.make_async_copy(
                        hw_hbm.at[src_v.at[pl.ds(cc * _CH, _CH)]],
                        stg_v.at[buf], sem).wait()

                _gat(0, 0, sem0)
                for cc in range(_SB // _CH):
                    buf = cc % 2
                    sem = sem0 if buf == 0 else sem1
                    if cc + 1 < _SB // _CH:
                        _gat(cc + 1, 1 - buf, sem1 if buf == 0 else sem0)
                    _wait(cc, buf, sem)
                    pltpu.sync_copy(
                        stg_v.at[buf], acc_sh.at[loc_v.at[cc]], add=True)
                return carry

            lax.fori_loop(0, n_sb, _sb_body, 0)
            plsc.subcore_barrier()
            pltpu.sync_copy(
                acc_sh.at[pl.ds(s * WPT, WPT)],
                out_hbm.at[c, pl.ds(base + s * WPT, WPT)])
            plsc.subcore_barrier()

    grid = (P // _BM,)
    f32 = jnp.float32

    def _row_spec():
        return pl.BlockSpec((_BM, D), lambda i: (i, 0))

    def _part_spec():
        return pl.BlockSpec((_NC, _BM, D), lambda i: (0, i, 0))

    def _dis_spec():
        return pl.BlockSpec((_BM, 1), lambda i: (i, 0))

    def _full_spec(shape):
        return pl.BlockSpec(shape, lambda i: tuple(0 for _ in shape))

    def _prep_body(degp_ref, x_ref, w_ref, hw_ref, dis_ref):
        i = pl.program_id(0)
        deg = jnp.sum(degp_ref[0] + degp_ref[1], axis=1) + 1.0
        dis = lax.rsqrt(jnp.maximum(deg, 1.0))[:, None]
        rows = i * _BM + lax.broadcasted_iota(jnp.int32, (_BM, 1), 0)
        dis = jnp.where(rows < N, dis, 0.0)
        dis_ref[...] = dis
        hw_ref[...] = jnp.dot(x_ref[...], w_ref[...],
                              preferred_element_type=f32) * dis

    _prep = pl.pallas_call(
        _prep_body,
        grid=grid,
        in_specs=[_part_spec(), _row_spec(), _full_spec((D, D))],
        out_specs=[_row_spec(), _dis_spec()],
        out_shape=[jax.ShapeDtypeStruct((P, D), f32),
                   jax.ShapeDtypeStruct((P, 1), f32)],
    )

    def _elu(a):
        return jnp.where(a > 0, a, jnp.exp(jnp.minimum(a, 0.0)) - 1.0)

    def _mid_first_body(p_ref, hw_ref, dis_ref, b_ref, w_ref, hwn_ref, m_ref):
        dis = dis_ref[...]
        agg = dis * (p_ref[0] + p_ref[1] + hw_ref[...]) + b_ref[...]
        h = _elu(agg)
        m_ref[...] = h
        hwn_ref[...] = jnp.dot(h, w_ref[...], preferred_element_type=f32) * dis

    _mid_first = pl.pallas_call(
        _mid_first_body,
        grid=grid,
        in_specs=[_part_spec(), _row_spec(), _dis_spec(),
                  _full_spec((1, D)), _full_spec((D, D))],
        out_specs=[_row_spec(), _row_spec()],
        out_shape=[jax.ShapeDtypeStruct((P, D), f32),
                   jax.ShapeDtypeStruct((P, D), f32)],
    )

    def _mid_body(p_ref, hw_ref, dis_ref, b_ref, w_ref, m_ref,
                  hwn_ref, mo_ref):
        dis = dis_ref[...]
        agg = dis * (p_ref[0] + p_ref[1] + hw_ref[...]) + b_ref[...]
        h = _elu(agg)
        mo_ref[...] = jnp.maximum(m_ref[...], h)
        hwn_ref[...] = jnp.dot(h, w_ref[...], preferred_element_type=f32) * dis

    _mid = pl.pallas_call(
        _mid_body,
        grid=grid,
        in_specs=[_part_spec(), _row_spec(), _dis_spec(),
                  _full_spec((1, D)), _full_spec((D, D)), _row_spec()],
        out_specs=[_row_spec(), _row_spec()],
        out_shape=[jax.ShapeDtypeStruct((P, D), f32),
                   jax.ShapeDtypeStruct((P, D), f32)],
    )

    def _fin_body(p_ref, hw_ref, dis_ref, b_ref, m_ref, out_ref):
        agg = dis_ref[...] * (p_ref[0] + p_ref[1] + hw_ref[...]) + b_ref[...]
        out_ref[...] = jnp.maximum(m_ref[...], agg)

    _fin = pl.pallas_call(
        _fin_body,
        grid=grid,
        in_specs=[_part_spec(), _row_spec(), _dis_spec(),
                  _full_spec((1, D)), _row_spec()],
        out_specs=_row_spec(),
        out_shape=jax.ShapeDtypeStruct((P, D), f32),
    )

    return P, EPAD, _segsum, _prep, _mid_first, _mid, _fin


def kernel(x, edge_index, W, b):
    N, D = x.shape
    K = W.shape[0]
    E = edge_index.shape[1]
    P, EPAD, segsum, prep, mid_first, mid, fin = _build(N, D, E, K)
    HP = P // 2
    E8 = _round_up(E, 8)
    cap = E8 + _SB

    x_p = jnp.pad(x, ((0, P - N), (0, 0)))
    pad_idx = jnp.full((EPAD - E,), N, jnp.int32)
    src_e = jnp.concatenate([edge_index[0], pad_idx])
    dst_e = jnp.concatenate([edge_index[1], pad_idx])
    dst_s, src_s = lax.sort((dst_e, src_e), num_keys=1)

    hpb = jnp.searchsorted(dst_s, jnp.int32(HP)).astype(jnp.int32)
    wv = jnp.arange(_NW + 1, dtype=jnp.int32)
    ofs0 = ((hpb * wv) // _NW) & ~7
    ofs0 = ofs0.at[_NW].set(_round_up_i(hpb))
    len1 = jnp.int32(cap) - hpb
    ofs1 = ((hpb + (len1 * wv) // _NW) // 8) * 8
    ofs1 = ofs1.at[_NW].set(jnp.int32(cap))
    ofs = jnp.concatenate([ofs0, ofs1])
    ofs = jnp.pad(ofs, (0, _NOF - 2 * (_NW + 1)))
    onehot = jnp.zeros((P, D), jnp.float32).at[:, 0].set(1.0)

    degp = segsum(ofs, src_s, dst_s, onehot)
    hw, dis = prep(degp, x_p, W[0])
    m = None
    out = None
    for li in range(K):
        part = segsum(ofs, src_s, dst_s, hw)
        bi = b[li][None]
        if li == 0:
            hw, m = mid_first(part, hw, dis, bi, W[1])
        elif li < K - 1:
            hw, m = mid(part, hw, dis, bi, W[li + 1], m)
        else:
            out = fin(part, hw, dis, bi, m)
    return out[:N]


def _round_up_i(v):
    return ((v + 7) // 8) * 8
